# baseline (speedup 1.0000x reference)
import functools

import jax
import jax.numpy as jnp
from jax import lax
from jax.experimental import pallas as pl
from jax.experimental.pallas import tpu as pltpu
from jax.experimental.pallas import tpu_sc as plsc

_BS, _C1, _NY, _NX = 4, 96, 224, 224
_C2 = 96
_K = 3
_PY, _PX = _NY + 2, _NX + 2
_NW = 32
_CHUNK = 128
_BN = 512


def _sc_gather_build(n_pad):
    npw = n_pad // _NW
    nch = npw // _CHUNK
    mesh = plsc.VectorSubcoreMesh(core_axis_name="c", subcore_axis_name="s")

    @functools.partial(
        pl.kernel,
        mesh=mesh,
        out_type=jax.ShapeDtypeStruct((_K * _K, n_pad, _C1), jnp.float32),
        scratch_types=[
            pltpu.VMEM((npw,), jnp.int32),
            pltpu.VMEM((npw,), jnp.int32),
            pltpu.VMEM((npw,), jnp.int32),
            pltpu.VMEM((npw,), jnp.int32),
            pltpu.VMEM((_CHUNK,), jnp.int32),
            pltpu.VMEM((_CHUNK, _C1), jnp.float32),
            pltpu.SemaphoreType.DMA,
        ],
        compiler_params=pltpu.CompilerParams(use_tc_tiling_on_sc=False),
    )
    def sc_gather(table_hbm, bi_hbm, yi_hbm, xi_hbm, out_hbm,
                  bi_v, yi_v, xi_v, base_v, idx_v, rows_v, sem):
        ci = lax.axis_index("c")
        si = lax.axis_index("s")
        wid = si * 2 + ci
        pbase = wid * npw
        pltpu.sync_copy(bi_hbm.at[pl.ds(pbase, npw)], bi_v)
        pltpu.sync_copy(yi_hbm.at[pl.ds(pbase, npw)], yi_v)
        pltpu.sync_copy(xi_hbm.at[pl.ds(pbase, npw)], xi_v)

        def calc_base(j, carry):
            s = j * 16
            b16 = bi_v[pl.ds(s, 16)]
            y16 = yi_v[pl.ds(s, 16)]
            x16 = xi_v[pl.ds(s, 16)]
            base_v[pl.ds(s, 16)] = (b16 * _PY + y16) * _PX + x16
            return carry

        lax.fori_loop(0, npw // 16, calc_base, 0)

        for t in range(_K * _K):
            off = (t // _K) * _PX + (t % _K)

            def per_chunk(c, carry):
                def calc_idx(j, carry2):
                    s = j * 16
                    idx_v[pl.ds(s, 16)] = base_v[pl.ds(c * _CHUNK + s, 16)] + off
                    return carry2

                lax.fori_loop(0, _CHUNK // 16, calc_idx, 0)
                pltpu.async_copy(table_hbm.at[idx_v], rows_v, sem).wait()
                pltpu.sync_copy(
                    rows_v,
                    out_hbm.at[t, pl.ds(pbase + c * _CHUNK, _CHUNK)])
                return carry

            lax.fori_loop(0, nch, per_chunk, 0)

    return sc_gather


def _gemm_body(g_ref, w_ref, b_ref, o_ref):
    acc = jnp.broadcast_to(b_ref[...], (o_ref.shape[0], _C2))
    for t in range(_K * _K):
        acc = acc + jnp.dot(g_ref[t], w_ref[t],
                            preferred_element_type=jnp.float32)
    o_ref[...] = acc


def _tc_gemm(g3, w3, bias_row):
    n_pad = g3.shape[1]
    nt = _K * _K
    return pl.pallas_call(
        _gemm_body,
        grid=(n_pad // _BN,),
        in_specs=[
            pl.BlockSpec((nt, _BN, _C1), lambda i: (0, i, 0)),
            pl.BlockSpec((nt, _C1, _C2), lambda i: (0, 0, 0)),
            pl.BlockSpec((1, _C2), lambda i: (0, 0)),
        ],
        out_specs=pl.BlockSpec((_BN, _C2), lambda i: (i, 0)),
        out_shape=jax.ShapeDtypeStruct((n_pad, _C2), jnp.float32),
    )(g3, w3, bias_row)


def kernel(x, indices, weight_flatten, bias):
    n = indices.shape[0]
    n_pad = ((n + _NW * _CHUNK - 1) // (_NW * _CHUNK)) * (_NW * _CHUNK)

    xt = jnp.pad(jnp.transpose(x, (0, 2, 3, 1)),
                 ((0, 0), (1, 1), (1, 1), (0, 0)))
    table = xt.reshape(_BS * _PY * _PX, _C1)

    idx = indices.astype(jnp.int32)
    bi = jnp.pad(idx[:, 0], (0, n_pad - n))
    yi = jnp.pad(idx[:, 1], (0, n_pad - n))
    xi = jnp.pad(idx[:, 2], (0, n_pad - n))

    g3 = _sc_gather_build(n_pad)(table, bi, yi, xi)

    w3 = weight_flatten.reshape(_C2, _C1, _K * _K).transpose(2, 1, 0)

    z = _tc_gemm(g3, w3, bias.reshape(1, _C2))
    return z[:n]
